# Initial kernel scaffold; baseline (speedup 1.0000x reference)
import jax
import jax.numpy as jnp
from jax.experimental import pallas as pl


def kernel(x, edge_index, W0, b0, p0, W1, b1, p1, W2, b2, p2):
    raise NotImplementedError("write your pallas kernel here")


# speedup vs baseline: 3.5737x; 3.5737x over previous
import functools
import math

import jax
import jax.numpy as jnp
from jax.experimental import pallas as pl

_INTERPRET = False

N = 10000
NPAD = 10240
D = 128
E = 320000
NEG_SLOPE = 0.01
SIGN = -2147483648


def _pcall(body, out_shape, grid, in_specs, out_specs):
    return pl.pallas_call(
        body,
        out_shape=out_shape,
        grid=grid,
        in_specs=in_specs,
        out_specs=out_specs,
        interpret=_INTERPRET,
    )


def _k2_body(y_ref, vs_ref, degm_ref, w_ref, hp_ref, dis_ref):
    xr = y_ref[...] * vs_ref[...]
    h = jax.lax.dot_general(xr, w_ref[...], (((1,), (0,)), ((), ())),
                            preferred_element_type=jnp.float32)
    deg = jnp.sum(degm_ref[...], axis=1, keepdims=True)
    dis = jnp.where(deg > 0.0, jax.lax.rsqrt(deg), 0.0)
    hp_ref[...] = h * dis
    dis_ref[...] = dis


def _k2(y_prev, valsel, degm, W, bm=1024):
    nb = NPAD // bm
    dc = degm.shape[1]
    return _pcall(
        _k2_body,
        out_shape=(jax.ShapeDtypeStruct((NPAD, D), jnp.float32),
                   jax.ShapeDtypeStruct((NPAD, 1), jnp.float32)),
        grid=(nb,),
        in_specs=[
            pl.BlockSpec((bm, D), lambda i: (i, 0)),
            pl.BlockSpec((bm, 1), lambda i: (i, 0)),
            pl.BlockSpec((bm, dc), lambda i: (i, 0)),
            pl.BlockSpec((D, D), lambda i: (0, 0)),
        ],
        out_specs=(pl.BlockSpec((bm, D), lambda i: (i, 0)),
                   pl.BlockSpec((bm, 1), lambda i: (i, 0))),
    )(y_prev, valsel, degm, W)


def _k4a_body(hp_ref, s0_ref, s1_ref, dis_ref, b_ref, p_ref, y_ref, sc_ref):
    pre = b_ref[...] + dis_ref[...] * (hp_ref[...] + s0_ref[...] + s1_ref[...])
    y = jnp.where(pre >= 0.0, pre, NEG_SLOPE * pre)
    y_ref[...] = y
    p = p_ref[...]
    pnorm = jnp.sqrt(jnp.sum(p * p))
    sraw = jax.lax.dot_general(y, p, (((1,), (0,)), ((), ())),
                               preferred_element_type=jnp.float32)
    sc_ref[...] = jnp.tanh(sraw / pnorm)


def _k4a(hp, S0, S1, dis, b, p, bm=1024):
    nb = NPAD // bm
    return _pcall(
        _k4a_body,
        out_shape=(jax.ShapeDtypeStruct((NPAD, D), jnp.float32),
                   jax.ShapeDtypeStruct((NPAD, 1), jnp.float32)),
        grid=(nb,),
        in_specs=[
            pl.BlockSpec((bm, D), lambda i: (i, 0)),
            pl.BlockSpec((bm, D), lambda i: (i, 0)),
            pl.BlockSpec((bm, D), lambda i: (i, 0)),
            pl.BlockSpec((bm, 1), lambda i: (i, 0)),
            pl.BlockSpec((1, D), lambda i: (0, 0)),
            pl.BlockSpec((D, 1), lambda i: (0, 0)),
        ],
        out_specs=(pl.BlockSpec((bm, D), lambda i: (i, 0)),
                   pl.BlockSpec((bm, 1), lambda i: (i, 0))),
    )(hp, S0, S1, dis, b, p)


def _k4b_body(k, sc_ref, m_ref, sel_ref, vs_ref):
    score = sc_ref[...]
    m = m_ref[...]
    bits = jax.lax.bitcast_convert_type(score, jnp.int32)
    v = bits ^ jax.lax.shift_right_logical(
        jax.lax.shift_right_arithmetic(bits, 31), 1)
    v = jnp.where(m > 0.0, v, SIGN)

    def count_ge(vk):
        return jnp.sum((v >= vk).astype(jnp.int32))

    def body_u(i, tu):
        bit = jnp.left_shift(jnp.int32(1), 31 - i)
        cand = tu | bit
        cnt = count_ge(cand ^ SIGN)
        return jnp.where(cnt >= k, cand, tu)

    tu = jax.lax.fori_loop(0, 32, body_u, jnp.int32(0))
    vk = tu ^ SIGN

    n_gt = jnp.sum((v > vk).astype(jnp.int32))
    need = k - n_gt
    ties = v == vk
    rows = sc_ref.shape[0]
    idx = (jax.lax.broadcasted_iota(jnp.int32, (rows, 128), 0) * 128
           + jax.lax.broadcasted_iota(jnp.int32, (rows, 128), 1))

    def body_i(i, j0):
        cand = j0 | jnp.left_shift(jnp.int32(1), 13 - i)
        f = jnp.sum((ties & (idx < cand)).astype(jnp.int32))
        return jnp.where(f < need, cand, j0)

    j0 = jax.lax.fori_loop(0, 14, body_i, jnp.int32(0))

    sel = (v > vk) | (ties & (idx <= j0))
    self32 = sel.astype(jnp.float32)
    sel_ref[...] = self32
    vs_ref[...] = score * self32


def _k4b(score2d, m2d, k):
    rows = NPAD // 128
    return _pcall(
        functools.partial(_k4b_body, k),
        out_shape=(jax.ShapeDtypeStruct((rows, 128), jnp.float32),
                   jax.ShapeDtypeStruct((rows, 128), jnp.float32)),
        grid=(1,),
        in_specs=[pl.BlockSpec((rows, 128), lambda i: (0, 0)),
                  pl.BlockSpec((rows, 128), lambda i: (0, 0))],
        out_specs=(pl.BlockSpec((rows, 128), lambda i: (0, 0)),
                   pl.BlockSpec((rows, 128), lambda i: (0, 0))),
    )(score2d, m2d)


def _k4c_body(k, y_ref, vs_ref, sel_ref, f_ref):
    xn = y_ref[...] * vs_ref[...]
    selected = sel_ref[...] > 0.0
    gmax = jnp.max(jnp.where(selected, xn, -3.4e38), axis=0, keepdims=True)
    gmean = jnp.sum(xn, axis=0, keepdims=True) * (1.0 / k)
    f_ref[...] = jnp.concatenate([gmax, gmean], axis=1)


def _k4c(y, valsel, sel, k):
    return _pcall(
        functools.partial(_k4c_body, k),
        out_shape=jax.ShapeDtypeStruct((1, 2 * D), jnp.float32),
        grid=(1,),
        in_specs=[pl.BlockSpec((NPAD, D), lambda i: (0, 0)),
                  pl.BlockSpec((NPAD, 1), lambda i: (0, 0)),
                  pl.BlockSpec((NPAD, 1), lambda i: (0, 0))],
        out_specs=pl.BlockSpec((1, 2 * D), lambda i: (0, 0)),
    )(y, valsel, sel)


def _edge_work(m, row, col, hp):
    w = jnp.take(m, row)
    rowm = jnp.where(w > 0.0, row, N)
    deg = jnp.zeros((NPAD,), jnp.float32).at[col].add(w)
    S = jnp.zeros((NPAD, D), jnp.float32).at[col].add(jnp.take(hp, rowm, axis=0))
    return deg, S


def kernel(x, edge_index, W0, b0, p0, W1, b1, p1, W2, b2, p2):
    row = edge_index[0]
    col = edge_index[1]

    y = jnp.pad(x, ((0, NPAD - N), (0, 0)))
    m = (jnp.arange(NPAD) < N).astype(jnp.float32)
    valsel = m[:, None]

    n_cur = N
    feats = []
    zS = jnp.zeros((NPAD, D), jnp.float32)
    for (W, b, p) in ((W0, b0, p0), (W1, b1, p1), (W2, b2, p2)):
        k = math.ceil(0.5 * n_cur)
        w = jnp.take(m, row)
        deg = jnp.zeros((NPAD,), jnp.float32).at[col].add(w)
        degm = (deg + m)[:, None]
        hp, dis = _k2(y, valsel, degm, W)
        rowm = jnp.where(w > 0.0, row, N).astype(jnp.int32)
        S = jnp.zeros((NPAD, D), jnp.float32).at[col].add(
            jnp.take(hp, rowm, axis=0))
        y, score = _k4a(hp, S, zS, dis, b[None, :], p[:, None])
        sel2d, valsel2d = _k4b(score.reshape(NPAD // 128, 128),
                               m.reshape(NPAD // 128, 128), k)
        sel = sel2d.reshape(NPAD)
        valsel = valsel2d.reshape(NPAD, 1)
        feats.append(_k4c(y, valsel, sel[:, None], k))
        m = sel
        n_cur = k

    out = jnp.concatenate(feats, axis=1)
    return (out, jnp.zeros((), jnp.float32))
